# baseline (speedup 1.0000x reference)
import functools

import jax
import jax.numpy as jnp
from jax import lax
from jax.experimental import pallas as pl
from jax.experimental.pallas import tpu as pltpu
from jax.experimental.pallas import tpu_sc as plsc

BATCH = 4096
SEQ = 200
DIM = 32
VOCAB = 1000000

NUM_CORES = 2
NUM_SUBCORES = 16
NW = NUM_CORES * NUM_SUBCORES
BH = BATCH // 128
N_SLABS = SEQ * BH
SLABS_PER_W = N_SLABS // NW

_mesh = plsc.VectorSubcoreMesh(core_axis_name="c", subcore_axis_name="s")


@functools.partial(
    pl.kernel,
    mesh=_mesh,
    out_type=jax.ShapeDtypeStruct((SEQ, 4, BH, 8, 128), jnp.float32),
    scratch_types=[
        pltpu.VMEM((SLABS_PER_W, 128), jnp.int32),
        pltpu.VMEM((128, DIM), jnp.float32),
        pltpu.VMEM((128, DIM), jnp.float32),
        pltpu.VMEM((4, 8, 128), jnp.float32),
        pltpu.VMEM((4, 8, 128), jnp.float32),
        pltpu.SemaphoreType.DMA,
        pltpu.SemaphoreType.DMA,
        pltpu.SemaphoreType.DMA,
        pltpu.SemaphoreType.DMA,
    ],
    compiler_params=pltpu.CompilerParams(
        use_tc_tiling_on_sc=False, needs_layout_passes=False),
)
def _gather_kernel(
    idx_hbm, table_hbm, out_hbm,
    idx_v, rows0, rows1, tile0, tile1, sg0, sg1, so0, so1,
):
    wid = lax.axis_index("s") * NUM_CORES + lax.axis_index("c")
    base = wid * SLABS_PER_W
    pltpu.sync_copy(idx_hbm.at[wid], idx_v)

    rows = (rows0, rows1)
    tiles = (tile0, tile1)
    sgs = (sg0, sg1)
    sos = (so0, so1)

    pltpu.async_copy(table_hbm.at[idx_v.at[0]], rows0, sg0)
    pltpu.async_copy(table_hbm.at[idx_v.at[1]], rows1, sg1)

    def do_slab(i, b):
        gid = base + i
        s = gid >> 5
        bh = gid & 31
        iota16 = lax.iota(jnp.int32, 16)
        rvecs = [blk * 16 + iota16 for blk in range(8)]

        @pl.when(i >= 2)
        def _():
            for ch in range(4):
                pltpu.make_async_copy(
                    tiles[b].at[ch], out_hbm.at[0, ch, 0], sos[b]
                ).wait()

        pltpu.make_async_copy(
            table_hbm.at[idx_v.at[i]], rows[b], sgs[b]
        ).wait()

        for ch in range(4):
            for cl in range(8):
                cvec = jnp.full((16,), ch * 8 + cl, jnp.int32)
                for blk in range(8):
                    vals = plsc.load_gather(rows[b], [rvecs[blk], cvec])
                    tiles[b][ch, cl, pl.ds(blk * 16, 16)] = vals

        @pl.when(i + 2 < SLABS_PER_W)
        def _():
            pltpu.async_copy(table_hbm.at[idx_v.at[i + 2]], rows[b], sgs[b])

        for ch in range(4):
            pltpu.async_copy(tiles[b].at[ch], out_hbm.at[s, ch, bh], sos[b])

    def body(j, carry):
        do_slab(2 * j, 0)
        do_slab(2 * j + 1, 1)
        return carry

    lax.fori_loop(0, SLABS_PER_W // 2, body, 0)

    for b in range(2):
        for ch in range(4):
            pltpu.make_async_copy(
                tiles[b].at[ch], out_hbm.at[0, ch, 0], sos[b]
            ).wait()


def kernel(x, table):
    idx = x.T.reshape(NW, SLABS_PER_W, 128).astype(jnp.int32)
    out6d = _gather_kernel(idx, table)
    return out6d.transpose(2, 4, 0, 1, 3).reshape(BATCH, SEQ, DIM)


# speedup vs baseline: 1.4501x; 1.4501x over previous
import functools

import jax
import jax.numpy as jnp
from jax import lax
from jax.experimental import pallas as pl
from jax.experimental.pallas import tpu as pltpu
from jax.experimental.pallas import tpu_sc as plsc

BATCH = 4096
SEQ = 200
DIM = 32
VOCAB = 1000000
B = BATCH * SEQ

NUM_CORES = 2
NUM_SUBCORES = 16
NW = NUM_CORES * NUM_SUBCORES
B_PER_W = B // NW
CHUNK = 320
N_CHUNKS = B_PER_W // CHUNK

_mesh = plsc.VectorSubcoreMesh(core_axis_name="c", subcore_axis_name="s")


@functools.partial(
    pl.kernel,
    mesh=_mesh,
    out_type=jax.ShapeDtypeStruct((B, 128), jnp.float32),
    scratch_types=[
        pltpu.VMEM((N_CHUNKS, CHUNK), jnp.int32),
        pltpu.VMEM((CHUNK, 128), jnp.float32),
        pltpu.VMEM((CHUNK, 128), jnp.float32),
        pltpu.SemaphoreType.DMA,
        pltpu.SemaphoreType.DMA,
    ],
    compiler_params=pltpu.CompilerParams(use_tc_tiling_on_sc=False),
)
def _gather_kernel(idx_hbm, table_hbm, out_hbm, idx_v, rows0, rows1, sem0, sem1):
    wid = lax.axis_index("s") * NUM_CORES + lax.axis_index("c")
    base = wid * B_PER_W
    pltpu.sync_copy(idx_hbm.at[wid], idx_v)

    rows = (rows0, rows1)
    sems = (sem0, sem1)
    copies = [None, None]
    copies[0] = pltpu.async_copy(table_hbm.at[idx_v.at[0]], rows0, sem0)
    for i in range(N_CHUNKS):
        b = i % 2
        nb = (i + 1) % 2
        if i + 1 < N_CHUNKS:
            copies[nb] = pltpu.async_copy(
                table_hbm.at[idx_v.at[i + 1]], rows[nb], sems[nb]
            )
        copies[b].wait()
        pltpu.sync_copy(
            rows[b].at[:, pl.ds(0, DIM)],
            out_hbm.at[pl.ds(base + i * CHUNK, CHUNK), pl.ds(0, DIM)],
        )


def kernel(x, table):
    idx = x.reshape(NW, N_CHUNKS, CHUNK).astype(jnp.int32)
    table_p = jax.lax.optimization_barrier(
        jnp.pad(table, ((0, 0), (0, 128 - DIM))))
    out_pitched = _gather_kernel(idx, table_p)
    return out_pitched.reshape(BATCH, SEQ, 128)[:, :, :DIM]


# speedup vs baseline: 1.6623x; 1.1463x over previous
import functools

import jax
import jax.numpy as jnp
from jax import lax
from jax.experimental import pallas as pl
from jax.experimental.pallas import tpu as pltpu
from jax.experimental.pallas import tpu_sc as plsc

BATCH = 4096
SEQ = 200
DIM = 32
VOCAB = 1000000
B = BATCH * SEQ

NUM_CORES = 2
NUM_SUBCORES = 16
NW = NUM_CORES * NUM_SUBCORES
B_PER_W = B // NW
CHUNK = 1280
N_CHUNKS = B_PER_W // CHUNK

_mesh = plsc.VectorSubcoreMesh(core_axis_name="c", subcore_axis_name="s")


@functools.partial(
    pl.kernel,
    mesh=_mesh,
    out_type=jax.ShapeDtypeStruct((B, 128), jnp.float32),
    scratch_types=[
        pltpu.VMEM((N_CHUNKS, CHUNK), jnp.int32),
        pltpu.VMEM((CHUNK, DIM), jnp.float32),
        pltpu.VMEM((CHUNK, DIM), jnp.float32),
        pltpu.SemaphoreType.DMA,
        pltpu.SemaphoreType.DMA,
    ],
    compiler_params=pltpu.CompilerParams(use_tc_tiling_on_sc=False),
)
def _gather_kernel(idx_hbm, table_hbm, out_hbm, idx_v, rows0, rows1, sem0, sem1):
    wid = lax.axis_index("s") * NUM_CORES + lax.axis_index("c")
    base = wid * B_PER_W
    pltpu.sync_copy(idx_hbm.at[wid], idx_v)

    rows = (rows0, rows1)
    sems = (sem0, sem1)
    copies = [None, None]
    copies[0] = pltpu.async_copy(table_hbm.at[idx_v.at[0]], rows0, sem0)
    for i in range(N_CHUNKS):
        b = i % 2
        nb = (i + 1) % 2
        if i + 1 < N_CHUNKS:
            copies[nb] = pltpu.async_copy(
                table_hbm.at[idx_v.at[i + 1]], rows[nb], sems[nb]
            )
        copies[b].wait()
        pltpu.sync_copy(
            rows[b],
            out_hbm.at[pl.ds(base + i * CHUNK, CHUNK), pl.ds(0, DIM)],
        )


def kernel(x, table):
    idx = x.reshape(NW, N_CHUNKS, CHUNK).astype(jnp.int32)
    out_pitched = _gather_kernel(idx, table)
    return out_pitched.reshape(BATCH, SEQ, 128)[:, :, :DIM]


# speedup vs baseline: 1.6646x; 1.0014x over previous
import functools

import jax
import jax.numpy as jnp
from jax import lax
from jax.experimental import pallas as pl
from jax.experimental.pallas import tpu as pltpu
from jax.experimental.pallas import tpu_sc as plsc

BATCH = 4096
SEQ = 200
DIM = 32
VOCAB = 1000000
B = BATCH * SEQ

NUM_CORES = 2
NUM_SUBCORES = 16
NW = NUM_CORES * NUM_SUBCORES
B_PER_W = B // NW
CHUNK = 1600
N_CHUNKS = B_PER_W // CHUNK

_mesh = plsc.VectorSubcoreMesh(core_axis_name="c", subcore_axis_name="s")


@functools.partial(
    pl.kernel,
    mesh=_mesh,
    out_type=jax.ShapeDtypeStruct((B, 128), jnp.float32),
    scratch_types=[
        pltpu.VMEM((N_CHUNKS, CHUNK), jnp.int32),
        pltpu.VMEM((CHUNK, DIM), jnp.float32),
        pltpu.VMEM((CHUNK, DIM), jnp.float32),
        pltpu.SemaphoreType.DMA,
        pltpu.SemaphoreType.DMA,
    ],
    compiler_params=pltpu.CompilerParams(use_tc_tiling_on_sc=False),
)
def _gather_kernel(idx_hbm, table_hbm, out_hbm, idx_v, rows0, rows1, sem0, sem1):
    wid = lax.axis_index("s") * NUM_CORES + lax.axis_index("c")
    base = wid * B_PER_W
    pltpu.sync_copy(idx_hbm.at[wid], idx_v)

    rows = (rows0, rows1)
    sems = (sem0, sem1)
    copies = [None, None]
    copies[0] = pltpu.async_copy(table_hbm.at[idx_v.at[0]], rows0, sem0)
    for i in range(N_CHUNKS):
        b = i % 2
        nb = (i + 1) % 2
        if i + 1 < N_CHUNKS:
            copies[nb] = pltpu.async_copy(
                table_hbm.at[idx_v.at[i + 1]], rows[nb], sems[nb]
            )
        copies[b].wait()
        pltpu.sync_copy(
            rows[b],
            out_hbm.at[pl.ds(base + i * CHUNK, CHUNK), pl.ds(0, DIM)],
        )


def kernel(x, table):
    idx = x.reshape(NW, N_CHUNKS, CHUNK).astype(jnp.int32)
    out_pitched = _gather_kernel(idx, table)
    return out_pitched.reshape(BATCH, SEQ, 128)[:, :, :DIM]
